# baseline (speedup 1.0000x reference)
import jax
import jax.numpy as jnp
import numpy as np
from jax.experimental import pallas as pl

N = 10000; T = 50; M = 6; H = 128; NH = 8; DH = 16; PD = 5; F = 60
B = 1024
GRID = (N + B - 1) // B


def _traj_kernel(pxT_ref, pyT_ref, hdT_ref, ae_ref, We1b_ref, WKQ_ref,
                 We2KQ_ref, Wv_ref, We2V_ref, bv2_ref, WoP_ref, cmode_ref,
                 Wp2_ref, bp2_ref, out_ref):
    f32 = jnp.float32
    dn_t = (((0,), (0,)), ((), ()))

    def mm(a, b):
        return jnp.dot(a, b, preferred_element_type=f32)

    pxT = pxT_ref[...]
    pyT = pyT_ref[...]
    hdT = hdT_ref[...]
    xL = pxT[PD - 1:PD, :]
    yL = pyT[PD - 1:PD, :]
    hL = hdT[PD - 1:PD, :]
    ch = jnp.cos(hL)
    sh = jnp.sin(hL)
    rx = pxT - xL
    ry = pyT - yL
    lx = ch * rx + sh * ry
    ly = -sh * rx + ch * ry
    elen = jnp.sqrt(lx * lx + ly * ly + 1e-12)
    lxe = lx + 1e-6
    r2 = jnp.maximum(jnp.sqrt(lxe * lxe + ly * ly), 1e-30)
    sin_t = ly / r2
    cos_t = lxe / r2
    dh = hdT - hL
    sin_h = jnp.sin(dh)
    cos_h = jnp.cos(dh)

    rows = []
    for feat in (elen, sin_t, cos_t, sin_h, cos_h):
        rows.append(jnp.concatenate([feat[t:t + 1, :] for t in range(PD)],
                                    axis=1))
    itv = jnp.concatenate([jnp.full((1, B), float(t - PD), f32)
                           for t in range(PD)], axis=1)
    ones = jnp.full((1, PD * B), 1.0, f32)
    EA = jnp.concatenate(rows + [itv, ones], axis=0)

    rh = jax.nn.relu(jax.lax.dot_general(EA, We1b_ref[...], dn_t,
                                         preferred_element_type=f32))

    WKQ = WKQ_ref[...]
    We2KQ = We2KQ_ref[...]
    Wv = Wv_ref[...]
    We2V = We2V_ref[...]
    bv2 = bv2_ref[...]
    sc_t = []
    v_t = []
    for t in range(PD):
        ae_t = ae_ref[t]
        rh_t = rh[t * B:(t + 1) * B, :]
        sc_t.append(mm(ae_t, WKQ) + mm(rh_t, We2KQ))
        v_t.append(mm(ae_t, Wv) + mm(rh_t, We2V) + bv2)

    mx = sc_t[0]
    for t in range(1, PD):
        mx = jnp.maximum(mx, sc_t[t])
    ex = [jnp.exp(sc_t[t] - mx) for t in range(PD)]
    den = ex[0]
    for t in range(1, PD):
        den = den + ex[t]
    inv = 1.0 / (den + 1e-9)

    ri = jax.lax.broadcasted_iota(jnp.int32, (M * NH, M * H), 0)
    oi = jax.lax.broadcasted_iota(jnp.int32, (M * NH, M * H), 1)
    E = ((oi // H == ri // NH) & ((oi % H) // DH == ri % NH)).astype(f32)

    accs = [None] * M
    for t in range(PD):
        ax = mm(ex[t] * inv, E)
        for m in range(M):
            term = ax[:, m * H:(m + 1) * H] * v_t[t]
            accs[m] = term if accs[m] is None else accs[m] + term

    OW = 2 * F
    cmode = cmode_ref[...]
    Wp2 = Wp2_ref[...]
    bp2 = bp2_ref[...]
    WoP = WoP_ref[...]
    for m in range(M):
        h1 = jax.nn.relu(mm(accs[m], WoP) + cmode[m:m + 1, :])
        out_ref[:, m * OW:(m + 1) * OW] = mm(h1, Wp2) + bp2


def kernel(position, heading, a_embs, mode_w, We1, be1, We2, be2, Wq, bq,
           Wk, bk, Wv, bv, Wo, bo, Wp1, bp1, Wp2, bp2):
    pxT = position[:, T - PD:, 0].T
    pyT = position[:, T - PD:, 1].T
    hdT = heading[:, T - PD:].T
    ae5T = jnp.transpose(a_embs[:, T - PD:, :], (1, 0, 2))
    We1b = jnp.concatenate([We1, be1[None, :]], axis=0)

    qs = (mode_w @ Wq + bq) * 0.25
    cols = np.arange(M * NH)
    drows = np.arange(H)
    S48 = jnp.asarray((drows[:, None] // DH == cols[None, :] % NH),
                      dtype=jnp.float32)
    WS = qs.T[:, cols // NH] * S48
    WKQ = Wk @ WS
    We2KQ = We2 @ WKQ
    We2V = We2 @ Wv
    bv2 = (be2 @ Wv + bv).reshape(1, H)
    WoP = Wo @ Wp1
    cmode = mode_w @ Wp1 + (bo @ Wp1 + bp1)[None, :]

    out = pl.pallas_call(
        _traj_kernel,
        grid=(GRID,),
        in_specs=[
            pl.BlockSpec((PD, B), lambda i: (0, i)),
            pl.BlockSpec((PD, B), lambda i: (0, i)),
            pl.BlockSpec((PD, B), lambda i: (0, i)),
            pl.BlockSpec((PD, B, H), lambda i: (0, i, 0)),
            pl.BlockSpec((7, H), lambda i: (0, 0)),
            pl.BlockSpec((H, M * NH), lambda i: (0, 0)),
            pl.BlockSpec((H, M * NH), lambda i: (0, 0)),
            pl.BlockSpec((H, H), lambda i: (0, 0)),
            pl.BlockSpec((H, H), lambda i: (0, 0)),
            pl.BlockSpec((1, H), lambda i: (0, 0)),
            pl.BlockSpec((H, H), lambda i: (0, 0)),
            pl.BlockSpec((M, H), lambda i: (0, 0)),
            pl.BlockSpec((H, 2 * F), lambda i: (0, 0)),
            pl.BlockSpec((1, 2 * F), lambda i: (0, 0)),
        ],
        out_specs=pl.BlockSpec((B, M * 2 * F), lambda i: (i, 0)),
        out_shape=jax.ShapeDtypeStruct((N, M * 2 * F), jnp.float32),
    )(pxT, pyT, hdT, ae5T, We1b, WKQ, We2KQ, Wv, We2V, bv2, WoP, cmode,
      Wp2, bp2.reshape(1, 2 * F))
    return out.reshape(N, M, F, 2)
